# baseline (speedup 1.0000x reference)
import jax
import jax.numpy as jnp
from jax.experimental import pallas as pl


def _gcn_conv(h, ei, ew, W, b):
    n = h.shape[0]
    h = h @ W
    loop = jnp.arange(n)
    row = jnp.concatenate([ei[0], loop])
    col = jnp.concatenate([ei[1], loop])
    w = jnp.concatenate([ew, jnp.ones((n,), h.dtype)])
    deg = jax.ops.segment_sum(w, col, num_segments=n)
    dis = jax.lax.rsqrt(jnp.maximum(deg, 1e-12))
    norm = dis[row] * w * dis[col]
    out = jax.ops.segment_sum(h[row] * norm[:, None], col, num_segments=n)
    return out + b


def _mlp_body(mean_ref, mw1_ref, mb1_ref, mw2_ref, mb2_ref, out_ref):
    z = jnp.maximum(mean_ref[...] @ mw1_ref[...] + mb1_ref[...][None, :], 0.0)
    out_ref[...] = z @ mw2_ref[...] + mb2_ref[...][None, :]


def kernel(x, edge_index, edge_weight, batch, label_emb, W1, b1, W2, b2, mw1, mb1, mw2, mb2):
    h = label_emb[x]
    h = jax.nn.relu(_gcn_conv(h, edge_index, edge_weight, W1, b1))
    h = jax.nn.relu(_gcn_conv(h, edge_index, edge_weight, W2, b2))
    G = 512
    sums = jax.ops.segment_sum(h, batch, num_segments=G)
    cnt = jax.ops.segment_sum(jnp.ones((h.shape[0],), h.dtype), batch, num_segments=G)
    mean = sums / jnp.maximum(cnt, 1.0)[:, None]
    out2 = pl.pallas_call(
        _mlp_body,
        out_shape=jax.ShapeDtypeStruct((G, 1), jnp.float32),
    )(mean, mw1, mb1, mw2, mb2)
    return out2.squeeze(-1)


# speedup vs baseline: 8.6323x; 8.6323x over previous
import functools

import jax
import jax.numpy as jnp
from jax import lax
from jax.experimental import pallas as pl
from jax.experimental.pallas import tpu as pltpu
from jax.experimental.pallas import tpu_sc as plsc

N = 50000
E = 800000
L = 1000
H = 64
G = 512

NP = 53248
EP = 819200
NB = NP // 512
TPT_E = EP // 16
CPT_E = TPT_E // 128
BPT_E = CPT_E // 8
WPT_D = EP // 32
CPT_D = WPT_D // 128
RPT = NP // 16

_mesh = plsc.VectorSubcoreMesh(core_axis_name="c", subcore_axis_name="s")


@functools.partial(
    pl.kernel,
    out_type=(
        jax.ShapeDtypeStruct((2, NP, 8), jnp.float32),
        jax.ShapeDtypeStruct((NP, H), jnp.float32),
    ),
    mesh=_mesh,
    compiler_params=pltpu.CompilerParams(use_tc_tiling_on_sc=False),
    scratch_types=[
        pltpu.VMEM_SHARED((NP, 8), jnp.float32),
        pltpu.VMEM((CPT_D, 128), jnp.int32),
        pltpu.VMEM((1024, 8), jnp.float32),
        pltpu.VMEM((128,), jnp.int32),
        pltpu.VMEM((128, H), jnp.float32),
        pltpu.SemaphoreType.DMA,
    ],
)
def _sc_deg_xgather(col2d, w8d, x2d, t1, zd, deg_out, g1_out,
                    deg8, colb, wb8, xidx, grows, sem):
    core = lax.axis_index("c")
    sid = lax.axis_index("s")
    wid = core * 16 + sid

    pltpu.sync_copy(zd, deg8.at[pl.ds(sid * RPT, RPT)])
    plsc.subcore_barrier()

    pltpu.sync_copy(col2d.at[pl.ds(wid * CPT_D, CPT_D)], colb)

    def deg_block(bb, _):
        pltpu.sync_copy(w8d.at[pl.ds(wid * WPT_D + bb * 1024, 1024)], wb8)
        for j in range(8):
            pltpu.sync_copy(wb8.at[pl.ds(j * 128, 128)],
                            deg8.at[colb.at[bb * 8 + j]], add=True)
        return ()

    lax.fori_loop(0, CPT_D // 8, deg_block, ())
    plsc.subcore_barrier()
    pltpu.sync_copy(deg8.at[pl.ds(sid * RPT, RPT)],
                    deg_out.at[core, pl.ds(sid * RPT, RPT)])

    for k in range(13):
        r = wid * 13 + k
        pltpu.sync_copy(x2d.at[r, 0], xidx)
        pltpu.async_copy(t1.at[xidx], grows, sem).wait()
        pltpu.sync_copy(grows, g1_out.at[pl.ds(r * 128, 128)])


@functools.partial(
    pl.kernel,
    out_type=jax.ShapeDtypeStruct((2, NP, 32), jnp.float32),
    mesh=_mesh,
    compiler_params=pltpu.CompilerParams(use_tc_tiling_on_sc=False),
    scratch_types=[
        pltpu.VMEM_SHARED((NP, 32), jnp.float32),
        pltpu.VMEM((8, 128), jnp.int32),
        pltpu.VMEM((8, 128), jnp.int32),
        pltpu.VMEM((8, 128), jnp.float32),
        pltpu.VMEM((128,), jnp.int32),
        pltpu.VMEM((128, 32), jnp.float32),
        pltpu.SemaphoreType.DMA,
    ],
)
def _sc_edges(row2d, col2d, w2d, hstack, za, acc_out,
              acc, rowb, colb, wb, idxb, rows, sem):
    core = lax.axis_index("c")
    sid = lax.axis_index("s")
    coreoff = core * NP

    pltpu.sync_copy(za, acc.at[pl.ds(sid * RPT, RPT)])
    plsc.subcore_barrier()

    def big_block(bb, _):
        cb = sid * CPT_E + bb * 8
        pltpu.sync_copy(row2d.at[pl.ds(cb, 8)], rowb)
        pltpu.sync_copy(col2d.at[pl.ds(cb, 8)], colb)
        pltpu.sync_copy(w2d.at[pl.ds(cb, 8)], wb)
        for j in range(8):
            for g in range(8):
                idxb[pl.ds(16 * g, 16)] = rowb[j, pl.ds(16 * g, 16)] + coreoff
            pltpu.async_copy(hstack.at[idxb], rows, sem).wait()

            def scale16(g, _):
                wvec = wb[j, pl.ds(16 * g, 16)]
                for t in range(16):
                    e = 16 * g + t
                    w = jnp.full((16,), wvec[t], jnp.float32)
                    rows[e, pl.ds(0, 16)] = rows[e, pl.ds(0, 16)] * w
                    rows[e, pl.ds(16, 16)] = rows[e, pl.ds(16, 16)] * w
                return ()

            lax.fori_loop(0, 8, scale16, ())
            pltpu.sync_copy(rows, acc.at[colb.at[j]], add=True)
        return ()

    lax.fori_loop(0, BPT_E, big_block, ())
    plsc.subcore_barrier()
    pltpu.sync_copy(acc.at[pl.ds(sid * RPT, RPT)],
                    acc_out.at[core, pl.ds(sid * RPT, RPT)])


def _prep_body(emb_ref, w1_ref, out_ref):
    out_ref[...] = jnp.dot(emb_ref[...], w1_ref[...],
                           preferred_element_type=jnp.float32)


def _dis_from(deg_ref):
    d = deg_ref[0, :, 0:1] + deg_ref[1, :, 0:1] + 1.0
    return lax.rsqrt(jnp.maximum(d, 1e-12))


def _scale1_body(deg_ref, g1_ref, hh_ref):
    dis = _dis_from(deg_ref)
    hpp = dis * g1_ref[...]
    hh_ref[0] = hpp[:, :32]
    hh_ref[1] = hpp[:, 32:]


def _mid_body(deg_ref, acc_ref, hh1_ref, b1_ref, w2_ref, hh2_ref):
    dis = _dis_from(deg_ref)
    bh = b1_ref[...]
    h1 = jnp.maximum((acc_ref[...] + hh1_ref[...]) * dis + bh, 0.0)
    h1full = jnp.concatenate([h1[0], h1[1]], axis=1)
    g2 = jnp.dot(h1full, w2_ref[...], preferred_element_type=jnp.float32)
    hpp2 = dis * g2
    hh2_ref[0] = hpp2[:, :32]
    hh2_ref[1] = hpp2[:, 32:]


def _final_body(deg_ref, acc_ref, hh2_ref, b2_ref, batch_ref,
                mw1_ref, mb1_ref, mw2t_ref, mb2_ref, out_ref,
                sums_ref, cnt_ref):
    i = pl.program_id(0)

    @pl.when(i == 0)
    def _():
        sums_ref[...] = jnp.zeros_like(sums_ref)
        cnt_ref[...] = jnp.zeros_like(cnt_ref)

    dis = _dis_from(deg_ref)
    bh = b2_ref[...]
    h2 = jnp.maximum((acc_ref[...] + hh2_ref[...]) * dis + bh, 0.0)
    h2full = jnp.concatenate([h2[0], h2[1]], axis=1)
    bvec = batch_ref[0, 0, :]
    mask = (lax.broadcasted_iota(jnp.int32, (G, 512), 0)
            == bvec[None, :]).astype(jnp.float32)
    sums_ref[...] += jnp.dot(mask, h2full, preferred_element_type=jnp.float32)
    cnt_ref[...] += jnp.broadcast_to(
        jnp.sum(mask, axis=1, keepdims=True), (G, 8))

    @pl.when(i == NB - 1)
    def _():
        mean = sums_ref[...] / jnp.maximum(cnt_ref[:, 0:1], 1.0)
        z = jnp.maximum(
            jnp.dot(mean, mw1_ref[...], preferred_element_type=jnp.float32)
            + mb1_ref[...], 0.0)
        o = jnp.sum(z * mw2t_ref[...], axis=1, keepdims=True) + mb2_ref[0, 0]
        out_ref[...] = jnp.broadcast_to(o, (G, 8))


def kernel(x, edge_index, edge_weight, batch, label_emb, W1, b1, W2, b2,
           mw1, mb1, mw2, mb2):
    f32 = jnp.float32
    i32 = jnp.int32

    row = jnp.concatenate([edge_index[0].astype(i32),
                           jnp.zeros((EP - E,), i32)]).reshape(EP // 128, 128)
    col = jnp.concatenate([edge_index[1].astype(i32),
                           jnp.zeros((EP - E,), i32)]).reshape(EP // 128, 128)
    ew = jnp.concatenate([edge_weight,
                          jnp.zeros((EP - E,), f32)]).reshape(EP // 128, 128)
    x2d = jnp.concatenate([x.astype(i32),
                           jnp.zeros((NP - N,), i32)]).reshape(NP // 128, 1, 128)
    batch3 = jnp.concatenate([batch.astype(i32),
                              jnp.full((NP - N,), G, i32)]).reshape(NB, 1, 512)
    zd = jnp.zeros((RPT, 8), f32)
    za = jnp.zeros((RPT, 32), f32)

    t1 = pl.pallas_call(
        _prep_body,
        out_shape=jax.ShapeDtypeStruct((L, H), f32),
    )(label_emb, W1)

    ew8 = jnp.pad(ew.reshape(EP, 1), ((0, 0), (0, 7)))
    deg_out, g1 = _sc_deg_xgather(col, ew8, x2d, t1, zd)

    hh1 = pl.pallas_call(
        _scale1_body,
        grid=(NB,),
        in_specs=[
            pl.BlockSpec((2, 512, 8), lambda i: (0, i, 0)),
            pl.BlockSpec((512, H), lambda i: (i, 0)),
        ],
        out_specs=pl.BlockSpec((2, 512, 32), lambda i: (0, i, 0)),
        out_shape=jax.ShapeDtypeStruct((2, NP, 32), f32),
    )(deg_out, g1)

    acc1 = _sc_edges(row, col, ew, hh1.reshape(2 * NP, 32), za)

    hh2 = pl.pallas_call(
        _mid_body,
        grid=(NB,),
        in_specs=[
            pl.BlockSpec((2, 512, 8), lambda i: (0, i, 0)),
            pl.BlockSpec((2, 512, 32), lambda i: (0, i, 0)),
            pl.BlockSpec((2, 512, 32), lambda i: (0, i, 0)),
            pl.BlockSpec((2, 1, 32), lambda i: (0, 0, 0)),
            pl.BlockSpec((H, H), lambda i: (0, 0)),
        ],
        out_specs=pl.BlockSpec((2, 512, 32), lambda i: (0, i, 0)),
        out_shape=jax.ShapeDtypeStruct((2, NP, 32), f32),
    )(deg_out, acc1, hh1, b1.reshape(2, 1, 32), W2)

    acc2 = _sc_edges(row, col, ew, hh2.reshape(2 * NP, 32), za)

    out8 = pl.pallas_call(
        _final_body,
        grid=(NB,),
        in_specs=[
            pl.BlockSpec((2, 512, 8), lambda i: (0, i, 0)),
            pl.BlockSpec((2, 512, 32), lambda i: (0, i, 0)),
            pl.BlockSpec((2, 512, 32), lambda i: (0, i, 0)),
            pl.BlockSpec((2, 1, 32), lambda i: (0, 0, 0)),
            pl.BlockSpec((1, 1, 512), lambda i: (i, 0, 0)),
            pl.BlockSpec((H, H), lambda i: (0, 0)),
            pl.BlockSpec((1, H), lambda i: (0, 0)),
            pl.BlockSpec((1, H), lambda i: (0, 0)),
            pl.BlockSpec((1, 8), lambda i: (0, 0)),
        ],
        out_specs=pl.BlockSpec((G, 8), lambda i: (0, 0)),
        out_shape=jax.ShapeDtypeStruct((G, 8), f32),
        scratch_shapes=[
            pltpu.VMEM((G, H), f32),
            pltpu.VMEM((G, 8), f32),
        ],
    )(deg_out, acc2, hh2, b2.reshape(2, 1, 32), batch3,
      mw1, mb1.reshape(1, H), mw2.reshape(1, H),
      jnp.broadcast_to(mb2.reshape(1, 1), (1, 8)))

    return out8[:, 0]


# speedup vs baseline: 10.9053x; 1.2633x over previous
import functools

import jax
import jax.numpy as jnp
from jax import lax
from jax.experimental import pallas as pl
from jax.experimental.pallas import tpu as pltpu
from jax.experimental.pallas import tpu_sc as plsc

N = 50000
E = 800000
L = 1000
H = 64
G = 512

NP = 53248
EP = 819200
NB = NP // 512
TPT_E = EP // 16
CPT_E = TPT_E // 128
KE = 2
BPT_E = CPT_E // KE
WPT_D = EP // 32
CPT_D = WPT_D // 128
RPT = NP // 16

_mesh = plsc.VectorSubcoreMesh(core_axis_name="c", subcore_axis_name="s")


@functools.partial(
    pl.kernel,
    out_type=(
        jax.ShapeDtypeStruct((2, NP, 8), jnp.float32),
        jax.ShapeDtypeStruct((NP, H), jnp.float32),
    ),
    mesh=_mesh,
    compiler_params=pltpu.CompilerParams(use_tc_tiling_on_sc=False),
    scratch_types=[
        pltpu.VMEM_SHARED((NP, 8), jnp.float32),
        pltpu.VMEM((CPT_D, 128), jnp.int32),
        pltpu.VMEM((1024, 8), jnp.float32),
        pltpu.VMEM((128,), jnp.int32),
        pltpu.VMEM((128, H), jnp.float32),
        pltpu.SemaphoreType.DMA,
    ],
)
def _sc_deg_xgather(col2d, w8d, x2d, t1, zd, deg_out, g1_out,
                    deg8, colb, wb8, xidx, grows, sem):
    core = lax.axis_index("c")
    sid = lax.axis_index("s")
    wid = core * 16 + sid

    pltpu.sync_copy(zd, deg8.at[pl.ds(sid * RPT, RPT)])
    plsc.subcore_barrier()

    pltpu.sync_copy(col2d.at[pl.ds(wid * CPT_D, CPT_D)], colb)

    def deg_block(bb, _):
        pltpu.sync_copy(w8d.at[pl.ds(wid * WPT_D + bb * 1024, 1024)], wb8)
        for j in range(8):
            pltpu.sync_copy(wb8.at[pl.ds(j * 128, 128)],
                            deg8.at[colb.at[bb * 8 + j]], add=True)
        return ()

    lax.fori_loop(0, CPT_D // 8, deg_block, ())
    plsc.subcore_barrier()
    pltpu.sync_copy(deg8.at[pl.ds(sid * RPT, RPT)],
                    deg_out.at[core, pl.ds(sid * RPT, RPT)])

    for k in range(13):
        r = wid * 13 + k
        pltpu.sync_copy(x2d.at[r, 0], xidx)
        pltpu.async_copy(t1.at[xidx], grows, sem).wait()
        pltpu.sync_copy(grows, g1_out.at[pl.ds(r * 128, 128)])


@functools.partial(
    pl.kernel,
    out_type=jax.ShapeDtypeStruct((2, NP, 32), jnp.float32),
    mesh=_mesh,
    compiler_params=pltpu.CompilerParams(use_tc_tiling_on_sc=False),
    scratch_types=[
        pltpu.VMEM_SHARED((NP, 32), jnp.float32),
        pltpu.VMEM((2, KE, 128), jnp.int32),
        pltpu.VMEM((2, KE, 128), jnp.int32),
        pltpu.VMEM((2, KE, 128), jnp.float32),
        pltpu.VMEM((2, KE, 128), jnp.int32),
        pltpu.VMEM((2, KE, 128, 32), jnp.float32),
        pltpu.SemaphoreType.DMA,
        pltpu.SemaphoreType.DMA,
        pltpu.SemaphoreType.DMA,
    ],
)
def _sc_edges(row2d, col2d, w2d, hstack, za, acc_out,
              acc, rowb, colb, wb, idxb, rows, sem_l, sem_g, sem_s):
    core = lax.axis_index("c")
    sid = lax.axis_index("s")
    coreoff = core * NP

    pltpu.sync_copy(za, acc.at[pl.ds(sid * RPT, RPT)])
    plsc.subcore_barrier()

    def fire_loads(blk, p):
        cb = sid * CPT_E + blk * KE
        pltpu.async_copy(row2d.at[pl.ds(cb, KE)], rowb.at[p], sem_l)
        pltpu.async_copy(col2d.at[pl.ds(cb, KE)], colb.at[p], sem_l)
        pltpu.async_copy(w2d.at[pl.ds(cb, KE)], wb.at[p], sem_l)

    def wait_loads(blk, p):
        cb = sid * CPT_E + blk * KE
        pltpu.make_async_copy(row2d.at[pl.ds(cb, KE)], rowb.at[p], sem_l).wait()
        pltpu.make_async_copy(col2d.at[pl.ds(cb, KE)], colb.at[p], sem_l).wait()
        pltpu.make_async_copy(w2d.at[pl.ds(cb, KE)], wb.at[p], sem_l).wait()

    def build_and_fire_gathers(p):
        for j in range(KE):
            for g in range(8):
                idxb[p, j, pl.ds(16 * g, 16)] = (
                    rowb[p, j, pl.ds(16 * g, 16)] + coreoff)
            pltpu.async_copy(hstack.at[idxb.at[p, j]], rows.at[p, j], sem_g)

    def wait_gathers(p):
        for j in range(KE):
            pltpu.make_async_copy(hstack.at[idxb.at[p, j]], rows.at[p, j],
                                  sem_g).wait()

    fire_loads(0, 0)
    wait_loads(0, 0)
    build_and_fire_gathers(0)
    fire_loads(1, 1)

    def half(b, p):
        wait_gathers(p)
        wait_loads(lax.rem(b + 1, BPT_E), 1 - p)
        build_and_fire_gathers(1 - p)
        sds = []
        for j in range(KE):
            def scale16(g, _):
                wvec = wb[p, j, pl.ds(16 * g, 16)]
                for t in range(16):
                    e = 16 * g + t
                    w = jnp.full((16,), wvec[t], jnp.float32)
                    rows[p, j, e, pl.ds(0, 16)] = rows[p, j, e, pl.ds(0, 16)] * w
                    rows[p, j, e, pl.ds(16, 16)] = rows[p, j, e, pl.ds(16, 16)] * w
                return ()

            lax.fori_loop(0, 8, scale16, ())
            sds.append(pltpu.async_copy(rows.at[p, j], acc.at[colb.at[p, j]],
                                        sem_s, add=True))
        for d in sds:
            d.wait()
        fire_loads(lax.rem(b + 2, BPT_E), p)

    def dbl(bb, _):
        half(2 * bb, 0)
        half(2 * bb + 1, 1)
        return ()

    lax.fori_loop(0, BPT_E // 2, dbl, ())
    wait_gathers(0)
    wait_loads(1, 1)
    plsc.subcore_barrier()
    pltpu.sync_copy(acc.at[pl.ds(sid * RPT, RPT)],
                    acc_out.at[core, pl.ds(sid * RPT, RPT)])


def _prep_body(emb_ref, w1_ref, out_ref):
    out_ref[...] = jnp.dot(emb_ref[...], w1_ref[...],
                           preferred_element_type=jnp.float32)


def _dis_from(deg_ref):
    d = deg_ref[0, :, 0:1] + deg_ref[1, :, 0:1] + 1.0
    return lax.rsqrt(jnp.maximum(d, 1e-12))


def _scale1_body(deg_ref, g1_ref, hh_ref):
    dis = _dis_from(deg_ref)
    hpp = dis * g1_ref[...]
    hh_ref[0] = hpp[:, :32]
    hh_ref[1] = hpp[:, 32:]


def _mid_body(deg_ref, acc_ref, hh1_ref, b1_ref, w2_ref, hh2_ref):
    dis = _dis_from(deg_ref)
    bh = b1_ref[...]
    h1 = jnp.maximum((acc_ref[...] + hh1_ref[...]) * dis + bh, 0.0)
    h1full = jnp.concatenate([h1[0], h1[1]], axis=1)
    g2 = jnp.dot(h1full, w2_ref[...], preferred_element_type=jnp.float32)
    hpp2 = dis * g2
    hh2_ref[0] = hpp2[:, :32]
    hh2_ref[1] = hpp2[:, 32:]


def _final_body(deg_ref, acc_ref, hh2_ref, b2_ref, batch_ref,
                mw1_ref, mb1_ref, mw2t_ref, mb2_ref, out_ref,
                sums_ref, cnt_ref):
    i = pl.program_id(0)

    @pl.when(i == 0)
    def _():
        sums_ref[...] = jnp.zeros_like(sums_ref)
        cnt_ref[...] = jnp.zeros_like(cnt_ref)

    dis = _dis_from(deg_ref)
    bh = b2_ref[...]
    h2 = jnp.maximum((acc_ref[...] + hh2_ref[...]) * dis + bh, 0.0)
    h2full = jnp.concatenate([h2[0], h2[1]], axis=1)
    bvec = batch_ref[0, 0, :]
    mask = (lax.broadcasted_iota(jnp.int32, (G, 512), 0)
            == bvec[None, :]).astype(jnp.float32)
    sums_ref[...] += jnp.dot(mask, h2full, preferred_element_type=jnp.float32)
    cnt_ref[...] += jnp.broadcast_to(
        jnp.sum(mask, axis=1, keepdims=True), (G, 8))

    @pl.when(i == NB - 1)
    def _():
        mean = sums_ref[...] / jnp.maximum(cnt_ref[:, 0:1], 1.0)
        z = jnp.maximum(
            jnp.dot(mean, mw1_ref[...], preferred_element_type=jnp.float32)
            + mb1_ref[...], 0.0)
        o = jnp.sum(z * mw2t_ref[...], axis=1, keepdims=True) + mb2_ref[0, 0]
        out_ref[...] = jnp.broadcast_to(o, (G, 8))


def kernel(x, edge_index, edge_weight, batch, label_emb, W1, b1, W2, b2,
           mw1, mb1, mw2, mb2):
    f32 = jnp.float32
    i32 = jnp.int32

    row = jnp.concatenate([edge_index[0].astype(i32),
                           jnp.zeros((EP - E,), i32)]).reshape(EP // 128, 128)
    col = jnp.concatenate([edge_index[1].astype(i32),
                           jnp.zeros((EP - E,), i32)]).reshape(EP // 128, 128)
    ew = jnp.concatenate([edge_weight,
                          jnp.zeros((EP - E,), f32)]).reshape(EP // 128, 128)
    x2d = jnp.concatenate([x.astype(i32),
                           jnp.zeros((NP - N,), i32)]).reshape(NP // 128, 1, 128)
    batch3 = jnp.concatenate([batch.astype(i32),
                              jnp.full((NP - N,), G, i32)]).reshape(NB, 1, 512)
    zd = jnp.zeros((RPT, 8), f32)
    za = jnp.zeros((RPT, 32), f32)

    t1 = pl.pallas_call(
        _prep_body,
        out_shape=jax.ShapeDtypeStruct((L, H), f32),
    )(label_emb, W1)

    ew8 = jnp.pad(ew.reshape(EP, 1), ((0, 0), (0, 7)))
    deg_out, g1 = _sc_deg_xgather(col, ew8, x2d, t1, zd)

    hh1 = pl.pallas_call(
        _scale1_body,
        grid=(NB,),
        in_specs=[
            pl.BlockSpec((2, 512, 8), lambda i: (0, i, 0)),
            pl.BlockSpec((512, H), lambda i: (i, 0)),
        ],
        out_specs=pl.BlockSpec((2, 512, 32), lambda i: (0, i, 0)),
        out_shape=jax.ShapeDtypeStruct((2, NP, 32), f32),
    )(deg_out, g1)

    acc1 = _sc_edges(row, col, ew, hh1.reshape(2 * NP, 32), za)

    hh2 = pl.pallas_call(
        _mid_body,
        grid=(NB,),
        in_specs=[
            pl.BlockSpec((2, 512, 8), lambda i: (0, i, 0)),
            pl.BlockSpec((2, 512, 32), lambda i: (0, i, 0)),
            pl.BlockSpec((2, 512, 32), lambda i: (0, i, 0)),
            pl.BlockSpec((2, 1, 32), lambda i: (0, 0, 0)),
            pl.BlockSpec((H, H), lambda i: (0, 0)),
        ],
        out_specs=pl.BlockSpec((2, 512, 32), lambda i: (0, i, 0)),
        out_shape=jax.ShapeDtypeStruct((2, NP, 32), f32),
    )(deg_out, acc1, hh1, b1.reshape(2, 1, 32), W2)

    acc2 = _sc_edges(row, col, ew, hh2.reshape(2 * NP, 32), za)

    out8 = pl.pallas_call(
        _final_body,
        grid=(NB,),
        in_specs=[
            pl.BlockSpec((2, 512, 8), lambda i: (0, i, 0)),
            pl.BlockSpec((2, 512, 32), lambda i: (0, i, 0)),
            pl.BlockSpec((2, 512, 32), lambda i: (0, i, 0)),
            pl.BlockSpec((2, 1, 32), lambda i: (0, 0, 0)),
            pl.BlockSpec((1, 1, 512), lambda i: (i, 0, 0)),
            pl.BlockSpec((H, H), lambda i: (0, 0)),
            pl.BlockSpec((1, H), lambda i: (0, 0)),
            pl.BlockSpec((1, H), lambda i: (0, 0)),
            pl.BlockSpec((1, 8), lambda i: (0, 0)),
        ],
        out_specs=pl.BlockSpec((G, 8), lambda i: (0, 0)),
        out_shape=jax.ShapeDtypeStruct((G, 8), f32),
        scratch_shapes=[
            pltpu.VMEM((G, H), f32),
            pltpu.VMEM((G, 8), f32),
        ],
    )(deg_out, acc2, hh2, b2.reshape(2, 1, 32), batch3,
      mw1, mb1.reshape(1, H), mw2.reshape(1, H),
      jnp.broadcast_to(mb2.reshape(1, 1), (1, 8)))

    return out8[:, 0]


# speedup vs baseline: 11.5800x; 1.0619x over previous
import functools

import jax
import jax.numpy as jnp
from jax import lax
from jax.experimental import pallas as pl
from jax.experimental.pallas import tpu as pltpu
from jax.experimental.pallas import tpu_sc as plsc

N = 50000
E = 800000
L = 1000
H = 64
G = 512

NP = 53248
EP = 819200
NB = NP // 2048
TB = 2048
TPT_E = EP // 16
CPT_E = TPT_E // 128
KE = 2
BPT_E = CPT_E // KE
WPT_D = EP // 32
CPT_D = WPT_D // 128
RPT = NP // 16

_mesh = plsc.VectorSubcoreMesh(core_axis_name="c", subcore_axis_name="s")


@functools.partial(
    pl.kernel,
    out_type=(
        jax.ShapeDtypeStruct((2, NP, 8), jnp.float32),
        jax.ShapeDtypeStruct((NP, H), jnp.float32),
    ),
    mesh=_mesh,
    compiler_params=pltpu.CompilerParams(use_tc_tiling_on_sc=False),
    scratch_types=[
        pltpu.VMEM_SHARED((NP, 8), jnp.float32),
        pltpu.VMEM((CPT_D, 128), jnp.int32),
        pltpu.VMEM((1024, 8), jnp.float32),
        pltpu.VMEM((128,), jnp.int32),
        pltpu.VMEM((128, H), jnp.float32),
        pltpu.SemaphoreType.DMA,
    ],
)
def _sc_deg_xgather(col2d, w8d, x2d, t1, zd, deg_out, g1_out,
                    deg8, colb, wb8, xidx, grows, sem):
    core = lax.axis_index("c")
    sid = lax.axis_index("s")
    wid = core * 16 + sid

    pltpu.sync_copy(zd, deg8.at[pl.ds(sid * RPT, RPT)])
    plsc.subcore_barrier()

    pltpu.sync_copy(col2d.at[pl.ds(wid * CPT_D, CPT_D)], colb)

    def deg_block(bb, _):
        pltpu.sync_copy(w8d.at[pl.ds(wid * WPT_D + bb * 1024, 1024)], wb8)
        for j in range(8):
            pltpu.sync_copy(wb8.at[pl.ds(j * 128, 128)],
                            deg8.at[colb.at[bb * 8 + j]], add=True)
        return ()

    lax.fori_loop(0, CPT_D // 8, deg_block, ())
    plsc.subcore_barrier()
    pltpu.sync_copy(deg8.at[pl.ds(sid * RPT, RPT)],
                    deg_out.at[core, pl.ds(sid * RPT, RPT)])

    for k in range(13):
        r = wid * 13 + k
        pltpu.sync_copy(x2d.at[r, 0], xidx)
        pltpu.async_copy(t1.at[xidx], grows, sem).wait()
        pltpu.sync_copy(grows, g1_out.at[pl.ds(r * 128, 128)])


@functools.partial(
    pl.kernel,
    out_type=jax.ShapeDtypeStruct((2, NP, 32), jnp.float32),
    mesh=_mesh,
    compiler_params=pltpu.CompilerParams(use_tc_tiling_on_sc=False),
    scratch_types=[
        pltpu.VMEM_SHARED((NP, 32), jnp.float32),
        pltpu.VMEM((2, KE, 128), jnp.int32),
        pltpu.VMEM((2, KE, 128), jnp.int32),
        pltpu.VMEM((2, KE, 128), jnp.float32),
        pltpu.VMEM((2, KE, 128), jnp.int32),
        pltpu.VMEM((2, KE, 128, 32), jnp.float32),
        pltpu.SemaphoreType.DMA,
        pltpu.SemaphoreType.DMA,
        pltpu.SemaphoreType.DMA,
    ],
)
def _sc_edges(row2d, col2d, w2d, hstack, za, acc_out,
              acc, rowb, colb, wb, idxb, rows, sem_l, sem_g, sem_s):
    core = lax.axis_index("c")
    sid = lax.axis_index("s")
    coreoff = core * NP

    pltpu.sync_copy(za, acc.at[pl.ds(sid * RPT, RPT)])
    plsc.subcore_barrier()

    def fire_loads(blk, p):
        cb = sid * CPT_E + blk * KE
        pltpu.async_copy(row2d.at[pl.ds(cb, KE)], rowb.at[p], sem_l)
        pltpu.async_copy(col2d.at[pl.ds(cb, KE)], colb.at[p], sem_l)
        pltpu.async_copy(w2d.at[pl.ds(cb, KE)], wb.at[p], sem_l)

    def wait_loads(blk, p):
        cb = sid * CPT_E + blk * KE
        pltpu.make_async_copy(row2d.at[pl.ds(cb, KE)], rowb.at[p], sem_l).wait()
        pltpu.make_async_copy(col2d.at[pl.ds(cb, KE)], colb.at[p], sem_l).wait()
        pltpu.make_async_copy(w2d.at[pl.ds(cb, KE)], wb.at[p], sem_l).wait()

    def build_and_fire_gathers(p):
        for j in range(KE):
            for g in range(8):
                idxb[p, j, pl.ds(16 * g, 16)] = (
                    rowb[p, j, pl.ds(16 * g, 16)] + coreoff)
            pltpu.async_copy(hstack.at[idxb.at[p, j]], rows.at[p, j], sem_g)

    def wait_gathers(p):
        for j in range(KE):
            pltpu.make_async_copy(hstack.at[idxb.at[p, j]], rows.at[p, j],
                                  sem_g).wait()

    fire_loads(0, 0)
    wait_loads(0, 0)
    build_and_fire_gathers(0)
    fire_loads(1, 1)

    def half(b, p):
        wait_gathers(p)
        wait_loads(lax.rem(b + 1, BPT_E), 1 - p)
        build_and_fire_gathers(1 - p)
        sds = []
        for j in range(KE):
            def scale16(g, _):
                wvec = wb[p, j, pl.ds(16 * g, 16)]
                for t in range(16):
                    e = 16 * g + t
                    w = jnp.full((16,), wvec[t], jnp.float32)
                    rows[p, j, e, pl.ds(0, 16)] = rows[p, j, e, pl.ds(0, 16)] * w
                    rows[p, j, e, pl.ds(16, 16)] = rows[p, j, e, pl.ds(16, 16)] * w
                return ()

            lax.fori_loop(0, 8, scale16, ())
            sds.append(pltpu.async_copy(rows.at[p, j], acc.at[colb.at[p, j]],
                                        sem_s, add=True))
        for d in sds:
            d.wait()
        fire_loads(lax.rem(b + 2, BPT_E), p)

    def dbl(bb, _):
        half(2 * bb, 0)
        half(2 * bb + 1, 1)
        return ()

    lax.fori_loop(0, BPT_E // 2, dbl, ())
    wait_gathers(0)
    wait_loads(1, 1)
    plsc.subcore_barrier()
    pltpu.sync_copy(acc.at[pl.ds(sid * RPT, RPT)],
                    acc_out.at[core, pl.ds(sid * RPT, RPT)])


def _prep_body(emb_ref, w1_ref, out_ref):
    out_ref[...] = jnp.dot(emb_ref[...], w1_ref[...],
                           preferred_element_type=jnp.float32)


def _dis_from(deg_ref):
    d = deg_ref[0, :, 0:1] + deg_ref[1, :, 0:1] + 1.0
    return lax.rsqrt(jnp.maximum(d, 1e-12))


def _scale1_body(deg_ref, g1_ref, hh_ref):
    dis = _dis_from(deg_ref)
    hpp = dis * g1_ref[...]
    hh_ref[0] = hpp[:, :32]
    hh_ref[1] = hpp[:, 32:]


def _mid_body(deg_ref, acc_ref, hh1_ref, b1_ref, w2_ref, hh2_ref):
    dis = _dis_from(deg_ref)
    bh = b1_ref[...]
    h1 = jnp.maximum((acc_ref[...] + hh1_ref[...]) * dis + bh, 0.0)
    h1full = jnp.concatenate([h1[0], h1[1]], axis=1)
    g2 = jnp.dot(h1full, w2_ref[...], preferred_element_type=jnp.float32)
    hpp2 = dis * g2
    hh2_ref[0] = hpp2[:, :32]
    hh2_ref[1] = hpp2[:, 32:]


def _final_body(deg_ref, acc_ref, hh2_ref, b2_ref, batch_ref,
                mw1_ref, mb1_ref, mw2t_ref, mb2_ref, out_ref,
                sums_ref, cnt_ref):
    i = pl.program_id(0)

    @pl.when(i == 0)
    def _():
        sums_ref[...] = jnp.zeros_like(sums_ref)
        cnt_ref[...] = jnp.zeros_like(cnt_ref)

    dis = _dis_from(deg_ref)
    bh = b2_ref[...]
    h2 = jnp.maximum((acc_ref[...] + hh2_ref[...]) * dis + bh, 0.0)
    h2full = jnp.concatenate([h2[0], h2[1]], axis=1)
    bvec = batch_ref[0, 0, :]
    mask = (lax.broadcasted_iota(jnp.int32, (G, TB), 0)
            == bvec[None, :]).astype(jnp.float32)
    sums_ref[...] += jnp.dot(mask, h2full, preferred_element_type=jnp.float32)
    cnt_ref[...] += jnp.broadcast_to(
        jnp.sum(mask, axis=1, keepdims=True), (G, 8))

    @pl.when(i == NB - 1)
    def _():
        mean = sums_ref[...] / jnp.maximum(cnt_ref[:, 0:1], 1.0)
        z = jnp.maximum(
            jnp.dot(mean, mw1_ref[...], preferred_element_type=jnp.float32)
            + mb1_ref[...], 0.0)
        o = jnp.sum(z * mw2t_ref[...], axis=1, keepdims=True) + mb2_ref[0, 0]
        out_ref[...] = jnp.broadcast_to(o, (G, 8))


def kernel(x, edge_index, edge_weight, batch, label_emb, W1, b1, W2, b2,
           mw1, mb1, mw2, mb2):
    f32 = jnp.float32
    i32 = jnp.int32

    row = jnp.concatenate([edge_index[0].astype(i32),
                           jnp.zeros((EP - E,), i32)]).reshape(EP // 128, 128)
    col = jnp.concatenate([edge_index[1].astype(i32),
                           jnp.zeros((EP - E,), i32)]).reshape(EP // 128, 128)
    ew = jnp.concatenate([edge_weight,
                          jnp.zeros((EP - E,), f32)]).reshape(EP // 128, 128)
    x2d = jnp.concatenate([x.astype(i32),
                           jnp.zeros((NP - N,), i32)]).reshape(NP // 128, 1, 128)
    batch3 = jnp.concatenate([batch.astype(i32),
                              jnp.full((NP - N,), G, i32)]).reshape(NB, 1, TB)
    zd = jnp.zeros((RPT, 8), f32)
    za = jnp.zeros((RPT, 32), f32)

    t1 = pl.pallas_call(
        _prep_body,
        out_shape=jax.ShapeDtypeStruct((L, H), f32),
    )(label_emb, W1)

    ew8 = jnp.pad(ew.reshape(EP, 1), ((0, 0), (0, 7)))
    deg_out, g1 = _sc_deg_xgather(col, ew8, x2d, t1, zd)

    hh1 = pl.pallas_call(
        _scale1_body,
        grid=(NB,),
        in_specs=[
            pl.BlockSpec((2, TB, 8), lambda i: (0, i, 0)),
            pl.BlockSpec((TB, H), lambda i: (i, 0)),
        ],
        out_specs=pl.BlockSpec((2, TB, 32), lambda i: (0, i, 0)),
        out_shape=jax.ShapeDtypeStruct((2, NP, 32), f32),
    )(deg_out, g1)

    acc1 = _sc_edges(row, col, ew, hh1.reshape(2 * NP, 32), za)

    hh2 = pl.pallas_call(
        _mid_body,
        grid=(NB,),
        in_specs=[
            pl.BlockSpec((2, TB, 8), lambda i: (0, i, 0)),
            pl.BlockSpec((2, TB, 32), lambda i: (0, i, 0)),
            pl.BlockSpec((2, TB, 32), lambda i: (0, i, 0)),
            pl.BlockSpec((2, 1, 32), lambda i: (0, 0, 0)),
            pl.BlockSpec((H, H), lambda i: (0, 0)),
        ],
        out_specs=pl.BlockSpec((2, TB, 32), lambda i: (0, i, 0)),
        out_shape=jax.ShapeDtypeStruct((2, NP, 32), f32),
    )(deg_out, acc1, hh1, b1.reshape(2, 1, 32), W2)

    acc2 = _sc_edges(row, col, ew, hh2.reshape(2 * NP, 32), za)

    out8 = pl.pallas_call(
        _final_body,
        grid=(NB,),
        in_specs=[
            pl.BlockSpec((2, TB, 8), lambda i: (0, i, 0)),
            pl.BlockSpec((2, TB, 32), lambda i: (0, i, 0)),
            pl.BlockSpec((2, TB, 32), lambda i: (0, i, 0)),
            pl.BlockSpec((2, 1, 32), lambda i: (0, 0, 0)),
            pl.BlockSpec((1, 1, TB), lambda i: (i, 0, 0)),
            pl.BlockSpec((H, H), lambda i: (0, 0)),
            pl.BlockSpec((1, H), lambda i: (0, 0)),
            pl.BlockSpec((1, H), lambda i: (0, 0)),
            pl.BlockSpec((1, 8), lambda i: (0, 0)),
        ],
        out_specs=pl.BlockSpec((G, 8), lambda i: (0, 0)),
        out_shape=jax.ShapeDtypeStruct((G, 8), f32),
        scratch_shapes=[
            pltpu.VMEM((G, H), f32),
            pltpu.VMEM((G, 8), f32),
        ],
    )(deg_out, acc2, hh2, b2.reshape(2, 1, 32), batch3,
      mw1, mb1.reshape(1, H), mw2.reshape(1, H),
      jnp.broadcast_to(mb2.reshape(1, 1), (1, 8)))

    return out8[:, 0]
